# baseline (speedup 1.0000x reference)
import functools

import jax
import jax.numpy as jnp
from jax import lax
from jax.experimental import pallas as pl
from jax.experimental.pallas import tpu as pltpu
from jax.experimental.pallas import tpu_sc as plsc

_EPS = 1e-5
_CHUNK = 128


def _sc_gather(table, idx2d, n_rows, d):
    info = plsc.get_sparse_core_info()
    nc, ns = info.num_cores, info.num_subcores
    nw = nc * ns
    n_chunks = idx2d.shape[0]
    chunks_per_w = n_chunks // nw
    rows_per_w = chunks_per_w * _CHUNK
    mesh = plsc.VectorSubcoreMesh(core_axis_name="c", subcore_axis_name="s")

    @functools.partial(
        pl.kernel,
        mesh=mesh,
        out_type=jax.ShapeDtypeStruct((n_rows, d), jnp.float32),
        scratch_types=[
            pltpu.VMEM((chunks_per_w, _CHUNK), jnp.int32),
            pltpu.VMEM((rows_per_w, d), jnp.float32),
            pltpu.SemaphoreType.DMA,
        ],
    )
    def k(table_hbm, idx_hbm, out_hbm, idx_v, rows_v, sem):
        wid = lax.axis_index("s") * nc + lax.axis_index("c")
        pltpu.sync_copy(idx_hbm.at[pl.ds(wid * chunks_per_w, chunks_per_w)], idx_v)
        copies = []
        for j in range(chunks_per_w):
            copies.append(
                pltpu.async_copy(
                    table_hbm.at[idx_v.at[j]],
                    rows_v.at[pl.ds(j * _CHUNK, _CHUNK)],
                    sem,
                )
            )
        for c in copies:
            c.wait()
        pltpu.sync_copy(rows_v, out_hbm.at[pl.ds(wid * rows_per_w, rows_per_w)])

    return k(table, idx2d)


def _tc_body(e_ref, w_ref, b_ref, p_ref, g_ref, bt_ref, o_ref):
    h = jax.lax.dot_general(
        e_ref[...], w_ref[...],
        dimension_numbers=(((1,), (0,)), ((), ())),
        preferred_element_type=jnp.float32,
        precision=jax.lax.Precision.DEFAULT,
    )
    h = h + b_ref[...] + p_ref[...]
    mean = jnp.mean(h, axis=-1, keepdims=True)
    c = h - mean
    var = jnp.mean(c * c, axis=-1, keepdims=True)
    o_ref[...] = c * jax.lax.rsqrt(var + _EPS) * g_ref[...] + bt_ref[...]


def kernel(x, tok_embed1, W2, b2, pos_embed, gamma, beta):
    batch, seq = x.shape
    vocab, embed = tok_embed1.shape
    hidden = W2.shape[1]
    n_rows = batch * seq

    idx2d = x.reshape(n_rows // _CHUNK, _CHUNK)
    e = _sc_gather(tok_embed1, idx2d, n_rows, embed)

    R = 256
    s_blks = seq // R

    out = pl.pallas_call(
        _tc_body,
        grid=(s_blks, batch),
        in_specs=[
            pl.BlockSpec((R, embed), lambda s, b: (b * s_blks + s, 0)),
            pl.BlockSpec((embed, hidden), lambda s, b: (0, 0)),
            pl.BlockSpec((1, hidden), lambda s, b: (0, 0)),
            pl.BlockSpec((R, hidden), lambda s, b: (s, 0)),
            pl.BlockSpec((1, hidden), lambda s, b: (0, 0)),
            pl.BlockSpec((1, hidden), lambda s, b: (0, 0)),
        ],
        out_specs=pl.BlockSpec((R, hidden), lambda s, b: (b * s_blks + s, 0)),
        out_shape=jax.ShapeDtypeStruct((n_rows, hidden), jnp.float32),
    )(
        e,
        W2,
        b2.reshape(1, hidden),
        pos_embed,
        gamma.reshape(1, hidden),
        beta.reshape(1, hidden),
    )
    return out.reshape(batch, seq, hidden)


# speedup vs baseline: 1.2551x; 1.2551x over previous
import functools

import jax
import jax.numpy as jnp
from jax import lax
from jax.experimental import pallas as pl
from jax.experimental.pallas import tpu as pltpu
from jax.experimental.pallas import tpu_sc as plsc

_EPS = 1e-5
_CHUNK = 128


def _sc_gather(table, idx2d, n_rows, d):
    info = plsc.get_sparse_core_info()
    nc, ns = info.num_cores, info.num_subcores
    nw = nc * ns
    n_chunks = idx2d.shape[0]
    chunks_per_w = n_chunks // nw
    rows_per_w = chunks_per_w * _CHUNK
    mesh = plsc.VectorSubcoreMesh(core_axis_name="c", subcore_axis_name="s")

    @functools.partial(
        pl.kernel,
        mesh=mesh,
        out_type=jax.ShapeDtypeStruct((n_rows, d), jnp.float32),
        scratch_types=[
            pltpu.VMEM((chunks_per_w, _CHUNK), jnp.int32),
            pltpu.VMEM((rows_per_w, d), jnp.float32),
            pltpu.SemaphoreType.DMA,
        ],
    )
    def k(table_hbm, idx_hbm, out_hbm, idx_v, rows_v, sem):
        wid = lax.axis_index("s") * nc + lax.axis_index("c")
        pltpu.sync_copy(idx_hbm.at[pl.ds(wid * chunks_per_w, chunks_per_w)], idx_v)
        copies = []
        for j in range(chunks_per_w):
            copies.append(
                pltpu.async_copy(
                    table_hbm.at[idx_v.at[j]],
                    rows_v.at[pl.ds(j * _CHUNK, _CHUNK)],
                    sem,
                )
            )
        for c in copies:
            c.wait()
        pltpu.sync_copy(rows_v, out_hbm.at[pl.ds(wid * rows_per_w, rows_per_w)])

    return k(table, idx2d)


def _tc_body(e_ref, w_ref, b_ref, p_ref, g_ref, bt_ref, o_ref):
    h = jax.lax.dot_general(
        e_ref[...], w_ref[...],
        dimension_numbers=(((1,), (0,)), ((), ())),
        preferred_element_type=jnp.float32,
        precision=jax.lax.Precision.DEFAULT,
    )
    h = h + b_ref[...] + p_ref[...]
    mean = jnp.mean(h, axis=-1, keepdims=True)
    c = h - mean
    var = jnp.mean(c * c, axis=-1, keepdims=True)
    o_ref[...] = c * jax.lax.rsqrt(var + _EPS) * g_ref[...] + bt_ref[...]


def kernel(x, tok_embed1, W2, b2, pos_embed, gamma, beta):
    batch, seq = x.shape
    vocab, embed = tok_embed1.shape
    hidden = W2.shape[1]
    n_rows = batch * seq

    idx2d = x.reshape(n_rows // _CHUNK, _CHUNK)
    e = _sc_gather(tok_embed1, idx2d, n_rows, embed)

    R = 1024
    s_blks = seq // R

    out = pl.pallas_call(
        _tc_body,
        grid=(s_blks, batch),
        in_specs=[
            pl.BlockSpec((R, embed), lambda s, b: (b * s_blks + s, 0)),
            pl.BlockSpec((embed, hidden), lambda s, b: (0, 0)),
            pl.BlockSpec((1, hidden), lambda s, b: (0, 0)),
            pl.BlockSpec((R, hidden), lambda s, b: (s, 0)),
            pl.BlockSpec((1, hidden), lambda s, b: (0, 0)),
            pl.BlockSpec((1, hidden), lambda s, b: (0, 0)),
        ],
        out_specs=pl.BlockSpec((R, hidden), lambda s, b: (b * s_blks + s, 0)),
        out_shape=jax.ShapeDtypeStruct((n_rows, hidden), jnp.float32),
    )(
        e,
        W2,
        b2.reshape(1, hidden),
        pos_embed,
        gamma.reshape(1, hidden),
        beta.reshape(1, hidden),
    )
    return out.reshape(batch, seq, hidden)


# speedup vs baseline: 1.2918x; 1.0293x over previous
import functools

import jax
import jax.numpy as jnp
from jax import lax
from jax.experimental import pallas as pl
from jax.experimental.pallas import tpu as pltpu
from jax.experimental.pallas import tpu_sc as plsc

_EPS = 1e-5
_CHUNK = 128


def _sc_gather(table, idx2d, n_rows, d):
    info = plsc.get_sparse_core_info()
    nc, ns = info.num_cores, info.num_subcores
    nw = nc * ns
    n_chunks = idx2d.shape[0]
    chunks_per_w = n_chunks // nw
    rows_per_w = chunks_per_w * _CHUNK
    mesh = plsc.VectorSubcoreMesh(core_axis_name="c", subcore_axis_name="s")

    @functools.partial(
        pl.kernel,
        mesh=mesh,
        out_type=jax.ShapeDtypeStruct((n_rows, d), jnp.float32),
        scratch_types=[
            pltpu.VMEM((chunks_per_w, _CHUNK), jnp.int32),
            pltpu.VMEM((rows_per_w, d), jnp.float32),
            pltpu.SemaphoreType.DMA,
        ],
    )
    def k(table_hbm, idx_hbm, out_hbm, idx_v, rows_v, sem):
        wid = lax.axis_index("s") * nc + lax.axis_index("c")
        pltpu.sync_copy(idx_hbm.at[pl.ds(wid * chunks_per_w, chunks_per_w)], idx_v)
        copies = []
        for j in range(chunks_per_w):
            copies.append(
                pltpu.async_copy(
                    table_hbm.at[idx_v.at[j]],
                    rows_v.at[pl.ds(j * _CHUNK, _CHUNK)],
                    sem,
                )
            )
        for c in copies:
            c.wait()
        pltpu.sync_copy(rows_v, out_hbm.at[pl.ds(wid * rows_per_w, rows_per_w)])

    return k(table, idx2d)


def _tc_body(e_ref, w_ref, b_ref, p_ref, g_ref, bt_ref, o_ref):
    h = jax.lax.dot_general(
        e_ref[...], w_ref[...],
        dimension_numbers=(((1,), (0,)), ((), ())),
        preferred_element_type=jnp.float32,
        precision=jax.lax.Precision.DEFAULT,
    )
    h = h + b_ref[...] + p_ref[...]
    mean = jnp.mean(h, axis=-1, keepdims=True)
    c = h - mean
    var = jnp.mean(c * c, axis=-1, keepdims=True)
    o_ref[...] = c * jax.lax.rsqrt(var + _EPS) * g_ref[...] + bt_ref[...]


def kernel(x, tok_embed1, W2, b2, pos_embed, gamma, beta):
    batch, seq = x.shape
    vocab, embed = tok_embed1.shape
    hidden = W2.shape[1]
    n_rows = batch * seq

    idx2d = x.reshape(n_rows // _CHUNK, _CHUNK)
    e = _sc_gather(tok_embed1, idx2d, n_rows, embed)

    R = 2048
    s_blks = seq // R

    out = pl.pallas_call(
        _tc_body,
        grid=(s_blks, batch),
        in_specs=[
            pl.BlockSpec((R, embed), lambda s, b: (b * s_blks + s, 0)),
            pl.BlockSpec((embed, hidden), lambda s, b: (0, 0)),
            pl.BlockSpec((1, hidden), lambda s, b: (0, 0)),
            pl.BlockSpec((R, hidden), lambda s, b: (s, 0)),
            pl.BlockSpec((1, hidden), lambda s, b: (0, 0)),
            pl.BlockSpec((1, hidden), lambda s, b: (0, 0)),
        ],
        out_specs=pl.BlockSpec((R, hidden), lambda s, b: (b * s_blks + s, 0)),
        out_shape=jax.ShapeDtypeStruct((n_rows, hidden), jnp.float32),
    )(
        e,
        W2,
        b2.reshape(1, hidden),
        pos_embed,
        gamma.reshape(1, hidden),
        beta.reshape(1, hidden),
    )
    return out.reshape(batch, seq, hidden)
